# baseline (speedup 1.0000x reference)
import functools

import jax
import jax.numpy as jnp
from jax import lax
from jax.experimental import pallas as pl
from jax.experimental.pallas import tpu as pltpu
from jax.experimental.pallas import tpu_sc as plsc

_N_GRAPHS = 64


def _sc_agg(h, src, dst, zeros):
    n, d = h.shape
    e = src.shape[0]
    nw = 32
    epw = e // nw
    chunk = 96
    n_main = epw // chunk
    etail = epw - n_main * chunk
    n_loop = (n_main - 8) // 4
    assert chunk % 16 == 0 and n_main == 4 * n_loop + 8 and etail % 8 == 0
    rpt = ((n // 16 + 7) // 8) * 8
    rtail = n - 15 * rpt

    mesh = plsc.VectorSubcoreMesh(core_axis_name="c", subcore_axis_name="s")

    vm = pltpu.VMEM
    scratch = []
    for _ in range(4):
        scratch += [vm((chunk,), jnp.int32),
                    vm((chunk,), jnp.int32),
                    vm((chunk,), jnp.int32),
                    vm((chunk, d), jnp.float32),
                    pltpu.SemaphoreType.DMA,
                    pltpu.SemaphoreType.DMA,
                    pltpu.SemaphoreType.DMA]
    scratch.append(vm((etail,), jnp.int32))
    scratch.append(pltpu.VMEM_SHARED((n, d), jnp.float32))

    @functools.partial(
        pl.kernel,
        out_type=jax.ShapeDtypeStruct((2 * n, d), jnp.float32),
        mesh=mesh,
        scratch_types=scratch,
    )
    def agg_kernel(h_hbm, src_hbm, dst_hbm, z_hbm, out_hbm, *rest):
        acc = rest[-1]
        didx_t = rest[-2]
        slots = [rest[7 * s:7 * s + 7] for s in range(4)]
        cid = lax.axis_index("c")
        sid = lax.axis_index("s")
        wid = sid * 2 + cid
        base = wid * epw

        def start_idx(s, ci):
            off = base + ci * chunk
            pltpu.async_copy(src_hbm.at[pl.ds(off, chunk)], slots[s][0], slots[s][4])
            pltpu.async_copy(dst_hbm.at[pl.ds(off, chunk)], slots[s][1], slots[s][4])

        def wait_idx(s):
            pltpu.make_async_copy(src_hbm.at[pl.ds(0, chunk)], slots[s][0],
                                  slots[s][4]).wait()
            pltpu.make_async_copy(dst_hbm.at[pl.ds(0, chunk)], slots[s][1],
                                  slots[s][4]).wait()

        def start_gather(s):
            pltpu.async_copy(h_hbm.at[slots[s][0]], slots[s][3], slots[s][5])

        def wait_gather(s):
            pltpu.make_async_copy(h_hbm.at[slots[s][0]], slots[s][3],
                                  slots[s][5]).wait()

        def start_scatter(s):
            pltpu.async_copy(slots[s][3], acc.at[slots[s][2]], slots[s][6],
                             add=True)

        def wait_scatter(s):
            pltpu.make_async_copy(slots[s][3], acc.at[slots[s][2]],
                                  slots[s][6]).wait()

        def step(c, s, wait_sc, wait_g_prev, prefetch, scatter_prev):
            prev = (s - 1) % 4
            wait_idx(s)
            if wait_sc:
                wait_scatter(s)
            for k in range(chunk // 16):
                slots[s][2][pl.ds(16 * k, 16)] = slots[s][1][pl.ds(16 * k, 16)]
            start_gather(s)
            if wait_g_prev:
                wait_gather(prev)
            if prefetch:
                start_idx((s + 3) % 4, c + 3)
            if scatter_prev:
                start_scatter(prev)

        @pl.when(sid < 15)
        def _():
            pltpu.sync_copy(z_hbm.at[pl.ds(sid * rpt, rpt)],
                            acc.at[pl.ds(sid * rpt, rpt)])

        @pl.when(sid == 15)
        def _():
            pltpu.sync_copy(z_hbm.at[pl.ds(15 * rpt, rtail)],
                            acc.at[pl.ds(15 * rpt, rtail)])

        plsc.subcore_barrier()

        start_idx(0, 0)
        start_idx(1, 1)
        start_idx(2, 2)
        step(0, 0, False, False, True, False)
        step(1, 1, False, True, True, True)
        step(2, 2, False, True, True, True)
        step(3, 3, False, True, True, True)

        def body(i, carry):
            c0 = 4 * i
            step(c0 + 0, 0, True, True, True, True)
            step(c0 + 1, 1, True, True, True, True)
            step(c0 + 2, 2, True, True, True, True)
            step(c0 + 3, 3, True, True, True, True)
            return carry

        lax.fori_loop(1, n_loop + 1, body, 0)
        c0 = 4 * (n_loop + 1)
        step(c0 + 0, 0, True, True, True, True)
        step(c0 + 1, 1, True, True, False, True)
        step(c0 + 2, 2, True, True, False, True)
        step(c0 + 3, 3, True, True, False, True)
        wait_gather(3)
        start_scatter(3)
        toff = base + n_main * chunk
        wait_scatter(0)
        pltpu.sync_copy(src_hbm.at[pl.ds(toff, etail)],
                        slots[0][0].at[pl.ds(0, etail)])
        pltpu.sync_copy(dst_hbm.at[pl.ds(toff, etail)], didx_t)
        pltpu.async_copy(h_hbm.at[slots[0][0].at[pl.ds(0, etail)]],
                         slots[0][3].at[pl.ds(0, etail), :], slots[0][5])
        pltpu.make_async_copy(h_hbm.at[slots[0][0].at[pl.ds(0, etail)]],
                              slots[0][3].at[pl.ds(0, etail), :],
                              slots[0][5]).wait()
        pltpu.sync_copy(slots[0][3].at[pl.ds(0, etail), :], acc.at[didx_t],
                        add=True)
        wait_scatter(1)
        wait_scatter(2)
        wait_scatter(3)

        plsc.subcore_barrier()

        @pl.when(sid < 15)
        def _():
            pltpu.sync_copy(acc.at[pl.ds(sid * rpt, rpt)],
                            out_hbm.at[pl.ds(cid * n + sid * rpt, rpt)])

        @pl.when(sid == 15)
        def _():
            pltpu.sync_copy(acc.at[pl.ds(15 * rpt, rtail)],
                            out_hbm.at[pl.ds(cid * n + 15 * rpt, rtail)])

    return agg_kernel(h, src, dst, zeros)


def _tc_root(h, Wo, br):
    n, d = h.shape
    blk = 1000
    nblk = n // blk

    def body(h_ref, wo_ref, br_ref, out_ref):
        out_ref[...] = jnp.dot(h_ref[...], wo_ref[...],
                               preferred_element_type=jnp.float32) + br_ref[...]

    return pl.pallas_call(
        body,
        grid=(nblk,),
        in_specs=[
            pl.BlockSpec((blk, d), lambda i: (i, 0)),
            pl.BlockSpec((d, d), lambda i: (0, 0)),
            pl.BlockSpec((1, d), lambda i: (0, 0)),
        ],
        out_specs=pl.BlockSpec((blk, d), lambda i: (i, 0)),
        out_shape=jax.ShapeDtypeStruct((n, d), jnp.float32),
    )(h, Wo, br.reshape(1, d))


def _tc_combine_root(p, r, Wr, Wo_next, br_next, n2):
    n, d = r.shape
    blk = 1000
    nblk = n // blk

    def body(p_ref, r_ref, wr_ref, wo_ref, br_ref, h_ref, rn_ref):
        s = p_ref[0] + p_ref[1]
        acc = jnp.dot(s, wr_ref[...], preferred_element_type=jnp.float32)
        h = jnp.maximum(acc + r_ref[...], 0.0)
        h_ref[...] = h
        rn_ref[...] = jnp.dot(h, wo_ref[...],
                              preferred_element_type=jnp.float32) + br_ref[...]

    return pl.pallas_call(
        body,
        grid=(nblk,),
        in_specs=[
            pl.BlockSpec((2, blk, d), lambda i: (0, i, 0)),
            pl.BlockSpec((blk, d), lambda i: (i, 0)),
            pl.BlockSpec((d, d), lambda i: (0, 0)),
            pl.BlockSpec((d, d), lambda i: (0, 0)),
            pl.BlockSpec((1, d), lambda i: (0, 0)),
        ],
        out_specs=[pl.BlockSpec((blk, d), lambda i: (i, 0)),
                   pl.BlockSpec((blk, d), lambda i: (i, 0))],
        out_shape=[jax.ShapeDtypeStruct((n, d), jnp.float32),
                   jax.ShapeDtypeStruct((n, d), jnp.float32)],
    )(p.reshape(2, n2, d), r, Wr, Wo_next, br_next.reshape(1, d))


def _tc_final(p, r, Wr, batch_col, bounds, W4, b4, W5, b5, n2):
    n, d = r.shape
    blk = 1000
    nblk = n // blk
    g = _N_GRAPHS
    h2 = W4.shape[1]
    nc = W5.shape[1]
    neg_inf = float("-inf")

    def body(p_ref, r_ref, wr_ref, bc_ref, bd_ref,
             w4_ref, b4_ref, w5_ref, b5_ref, out_ref, pooled):
        i = pl.program_id(0)
        acc = jnp.dot(p_ref[0] + p_ref[1], wr_ref[...],
                      preferred_element_type=jnp.float32)
        h3 = jnp.maximum(acc + r_ref[...], 0.0)

        @pl.when(i == 0)
        def _():
            pooled[...] = jnp.full((g, d), neg_inf, jnp.float32)

        g0 = bd_ref[0, 0, 0]
        g1 = bd_ref[0, 0, 1]

        def gbody(gi, carry):
            m = bc_ref[...] == gi
            cur = jnp.max(jnp.where(m, h3, neg_inf), axis=0, keepdims=True)
            pooled[pl.ds(gi, 1), :] = jnp.maximum(pooled[pl.ds(gi, 1), :], cur)
            return carry

        lax.fori_loop(g0, g1 + 1, gbody, 0)

        @pl.when(i == nblk - 1)
        def _():
            t = jnp.maximum(
                jnp.dot(pooled[...], w4_ref[...], preferred_element_type=jnp.float32)
                + b4_ref[...], 0.0)
            z = jnp.dot(t, w5_ref[...], preferred_element_type=jnp.float32) + b5_ref[...]
            mz = jnp.max(z, axis=-1, keepdims=True)
            ez = jnp.exp(z - mz)
            out_ref[...] = z - mz - jnp.log(jnp.sum(ez, axis=-1, keepdims=True))

    return pl.pallas_call(
        body,
        grid=(nblk,),
        in_specs=[
            pl.BlockSpec((2, blk, d), lambda i: (0, i, 0)),
            pl.BlockSpec((blk, d), lambda i: (i, 0)),
            pl.BlockSpec((d, d), lambda i: (0, 0)),
            pl.BlockSpec((blk, 1), lambda i: (i, 0)),
            pl.BlockSpec((1, 1, 2), lambda i: (i, 0, 0), memory_space=pltpu.SMEM),
            pl.BlockSpec((d, h2), lambda i: (0, 0)),
            pl.BlockSpec((1, h2), lambda i: (0, 0)),
            pl.BlockSpec((h2, nc), lambda i: (0, 0)),
            pl.BlockSpec((1, nc), lambda i: (0, 0)),
        ],
        out_specs=pl.BlockSpec((g, nc), lambda i: (0, 0)),
        out_shape=jax.ShapeDtypeStruct((g, nc), jnp.float32),
        scratch_shapes=[pltpu.VMEM((g, d), jnp.float32)],
    )(p.reshape(2, n2, d), r, Wr, batch_col, bounds,
      W4, b4.reshape(1, h2), W5, b5.reshape(1, nc))


def kernel(x, edge_index, batch, Wr1, br1, Wo1, Wr2, br2, Wo2, Wr3, br3, Wo3,
           W4, b4, W5, b5):
    n, d = x.shape
    src = edge_index[0].astype(jnp.int32)
    dst = edge_index[1].astype(jnp.int32)
    batch = batch.astype(jnp.int32)
    zeros = jnp.zeros((n, d), jnp.float32)
    blk = 1000
    bounds = jnp.stack([batch[::blk], batch[blk - 1::blk]], axis=1).reshape(-1, 1, 2)
    batch_col = batch.reshape(n, 1)

    r1 = _tc_root(x, Wo1, br1)
    p1 = _sc_agg(x, src, dst, zeros)
    h1, r2 = _tc_combine_root(p1, r1, Wr1, Wo2, br2, n)
    p2 = _sc_agg(h1, src, dst, zeros)
    h2, r3 = _tc_combine_root(p2, r2, Wr2, Wo3, br3, n)
    p3 = _sc_agg(h2, src, dst, zeros)
    return _tc_final(p3, r3, Wr3, batch_col, bounds, W4, b4, W5, b5, n)


# speedup vs baseline: 1.0787x; 1.0787x over previous
import functools

import jax
import jax.numpy as jnp
from jax import lax
from jax.experimental import pallas as pl
from jax.experimental.pallas import tpu as pltpu
from jax.experimental.pallas import tpu_sc as plsc

_N_GRAPHS = 64


def _sc_agg(h, src, dst, zeros):
    n, d = h.shape
    e = src.shape[0]
    nw = 32
    epw = e // nw
    chunk = 128
    n_main = epw // chunk
    etail = epw - n_main * chunk
    n_loop = (n_main - 6) // 3
    assert chunk % 16 == 0 and n_main == 3 * n_loop + 6 and etail % 8 == 0
    rpt = ((n // 16 + 7) // 8) * 8
    rtail = n - 15 * rpt

    mesh = plsc.VectorSubcoreMesh(core_axis_name="c", subcore_axis_name="s")

    vm = pltpu.VMEM
    scratch = []
    for _ in range(3):
        scratch += [vm((chunk,), jnp.int32),
                    vm((chunk,), jnp.int32),
                    vm((chunk,), jnp.int32),
                    vm((chunk, d), jnp.float32),
                    pltpu.SemaphoreType.DMA,
                    pltpu.SemaphoreType.DMA,
                    pltpu.SemaphoreType.DMA]
    scratch.append(vm((etail,), jnp.int32))
    scratch.append(pltpu.VMEM_SHARED((n, d), jnp.float32))

    @functools.partial(
        pl.kernel,
        out_type=jax.ShapeDtypeStruct((2 * n, d), jnp.float32),
        mesh=mesh,
        scratch_types=scratch,
    )
    def agg_kernel(h_hbm, src_hbm, dst_hbm, z_hbm, out_hbm, *rest):
        acc = rest[-1]
        didx_t = rest[-2]
        slots = [rest[7 * s:7 * s + 7] for s in range(3)]
        cid = lax.axis_index("c")
        sid = lax.axis_index("s")
        wid = sid * 2 + cid
        base = wid * epw

        def start_idx(s, ci):
            off = base + ci * chunk
            pltpu.async_copy(src_hbm.at[pl.ds(off, chunk)], slots[s][0], slots[s][4])
            pltpu.async_copy(dst_hbm.at[pl.ds(off, chunk)], slots[s][1], slots[s][4])

        def wait_idx(s):
            pltpu.make_async_copy(src_hbm.at[pl.ds(0, chunk)], slots[s][0],
                                  slots[s][4]).wait()
            pltpu.make_async_copy(dst_hbm.at[pl.ds(0, chunk)], slots[s][1],
                                  slots[s][4]).wait()

        def start_gather(s):
            pltpu.async_copy(h_hbm.at[slots[s][0]], slots[s][3], slots[s][5])

        def wait_gather(s):
            pltpu.make_async_copy(h_hbm.at[slots[s][0]], slots[s][3],
                                  slots[s][5]).wait()

        def start_scatter(s):
            pltpu.async_copy(slots[s][3], acc.at[slots[s][2]], slots[s][6],
                             add=True)

        def wait_scatter(s):
            pltpu.make_async_copy(slots[s][3], acc.at[slots[s][2]],
                                  slots[s][6]).wait()

        def step(c, s, wait_sc, wait_g_prev, prefetch, scatter_prev):
            prev = (s - 1) % 3
            wait_idx(s)
            if wait_sc:
                wait_scatter(s)
            for k in range(chunk // 16):
                slots[s][2][pl.ds(16 * k, 16)] = slots[s][1][pl.ds(16 * k, 16)]
            start_gather(s)
            if wait_g_prev:
                wait_gather(prev)
            if prefetch:
                start_idx((s + 2) % 3, c + 2)
            if scatter_prev:
                start_scatter(prev)

        @pl.when(sid < 15)
        def _():
            pltpu.sync_copy(z_hbm.at[pl.ds(sid * rpt, rpt)],
                            acc.at[pl.ds(sid * rpt, rpt)])

        @pl.when(sid == 15)
        def _():
            pltpu.sync_copy(z_hbm.at[pl.ds(15 * rpt, rtail)],
                            acc.at[pl.ds(15 * rpt, rtail)])

        plsc.subcore_barrier()

        start_idx(0, 0)
        start_idx(1, 1)
        step(0, 0, False, False, True, False)
        step(1, 1, False, True, True, True)
        step(2, 2, False, True, True, True)

        def body(i, carry):
            c0 = 3 * i
            step(c0 + 0, 0, True, True, True, True)
            step(c0 + 1, 1, True, True, True, True)
            step(c0 + 2, 2, True, True, True, True)
            return carry

        lax.fori_loop(1, n_loop + 1, body, 0)
        c0 = 3 * (n_loop + 1)
        step(c0 + 0, 0, True, True, True, True)
        step(c0 + 1, 1, True, True, False, True)
        step(c0 + 2, 2, True, True, False, True)
        wait_gather(2)
        start_scatter(2)
        toff = base + n_main * chunk
        wait_scatter(0)
        pltpu.sync_copy(src_hbm.at[pl.ds(toff, etail)],
                        slots[0][0].at[pl.ds(0, etail)])
        pltpu.sync_copy(dst_hbm.at[pl.ds(toff, etail)], didx_t)
        pltpu.async_copy(h_hbm.at[slots[0][0].at[pl.ds(0, etail)]],
                         slots[0][3].at[pl.ds(0, etail), :], slots[0][5])
        pltpu.make_async_copy(h_hbm.at[slots[0][0].at[pl.ds(0, etail)]],
                              slots[0][3].at[pl.ds(0, etail), :],
                              slots[0][5]).wait()
        pltpu.sync_copy(slots[0][3].at[pl.ds(0, etail), :], acc.at[didx_t],
                        add=True)
        wait_scatter(1)
        wait_scatter(2)

        plsc.subcore_barrier()

        @pl.when(sid < 15)
        def _():
            pltpu.sync_copy(acc.at[pl.ds(sid * rpt, rpt)],
                            out_hbm.at[pl.ds(cid * n + sid * rpt, rpt)])

        @pl.when(sid == 15)
        def _():
            pltpu.sync_copy(acc.at[pl.ds(15 * rpt, rtail)],
                            out_hbm.at[pl.ds(cid * n + 15 * rpt, rtail)])

    return agg_kernel(h, src, dst, zeros)


def _tc_root(h, Wo, br):
    n, d = h.shape
    blk = 1000
    nblk = n // blk

    def body(h_ref, wo_ref, br_ref, out_ref):
        out_ref[...] = jnp.dot(h_ref[...], wo_ref[...],
                               preferred_element_type=jnp.float32) + br_ref[...]

    return pl.pallas_call(
        body,
        grid=(nblk,),
        in_specs=[
            pl.BlockSpec((blk, d), lambda i: (i, 0)),
            pl.BlockSpec((d, d), lambda i: (0, 0)),
            pl.BlockSpec((1, d), lambda i: (0, 0)),
        ],
        out_specs=pl.BlockSpec((blk, d), lambda i: (i, 0)),
        out_shape=jax.ShapeDtypeStruct((n, d), jnp.float32),
    )(h, Wo, br.reshape(1, d))


def _tc_combine_root(p, r, Wr, Wo_next, br_next, n2):
    n, d = r.shape
    blk = 1000
    nblk = n // blk

    def body(p_ref, r_ref, wr_ref, wo_ref, br_ref, h_ref, rn_ref):
        s = p_ref[0] + p_ref[1]
        acc = jnp.dot(s, wr_ref[...], preferred_element_type=jnp.float32)
        h = jnp.maximum(acc + r_ref[...], 0.0)
        h_ref[...] = h
        rn_ref[...] = jnp.dot(h, wo_ref[...],
                              preferred_element_type=jnp.float32) + br_ref[...]

    return pl.pallas_call(
        body,
        grid=(nblk,),
        in_specs=[
            pl.BlockSpec((2, blk, d), lambda i: (0, i, 0)),
            pl.BlockSpec((blk, d), lambda i: (i, 0)),
            pl.BlockSpec((d, d), lambda i: (0, 0)),
            pl.BlockSpec((d, d), lambda i: (0, 0)),
            pl.BlockSpec((1, d), lambda i: (0, 0)),
        ],
        out_specs=[pl.BlockSpec((blk, d), lambda i: (i, 0)),
                   pl.BlockSpec((blk, d), lambda i: (i, 0))],
        out_shape=[jax.ShapeDtypeStruct((n, d), jnp.float32),
                   jax.ShapeDtypeStruct((n, d), jnp.float32)],
    )(p.reshape(2, n2, d), r, Wr, Wo_next, br_next.reshape(1, d))


def _tc_final(p, r, Wr, batch_col, bounds, W4, b4, W5, b5, n2):
    n, d = r.shape
    blk = 1000
    nblk = n // blk
    g = _N_GRAPHS
    h2 = W4.shape[1]
    nc = W5.shape[1]
    neg_inf = float("-inf")

    def body(p_ref, r_ref, wr_ref, bc_ref, bd_ref,
             w4_ref, b4_ref, w5_ref, b5_ref, out_ref, pooled):
        i = pl.program_id(0)
        acc = jnp.dot(p_ref[0] + p_ref[1], wr_ref[...],
                      preferred_element_type=jnp.float32)
        h3 = jnp.maximum(acc + r_ref[...], 0.0)

        @pl.when(i == 0)
        def _():
            pooled[...] = jnp.full((g, d), neg_inf, jnp.float32)

        g0 = bd_ref[0, 0, 0]
        g1 = bd_ref[0, 0, 1]

        def gbody(gi, carry):
            m = bc_ref[...] == gi
            cur = jnp.max(jnp.where(m, h3, neg_inf), axis=0, keepdims=True)
            pooled[pl.ds(gi, 1), :] = jnp.maximum(pooled[pl.ds(gi, 1), :], cur)
            return carry

        lax.fori_loop(g0, g1 + 1, gbody, 0)

        @pl.when(i == nblk - 1)
        def _():
            t = jnp.maximum(
                jnp.dot(pooled[...], w4_ref[...], preferred_element_type=jnp.float32)
                + b4_ref[...], 0.0)
            z = jnp.dot(t, w5_ref[...], preferred_element_type=jnp.float32) + b5_ref[...]
            mz = jnp.max(z, axis=-1, keepdims=True)
            ez = jnp.exp(z - mz)
            out_ref[...] = z - mz - jnp.log(jnp.sum(ez, axis=-1, keepdims=True))

    return pl.pallas_call(
        body,
        grid=(nblk,),
        in_specs=[
            pl.BlockSpec((2, blk, d), lambda i: (0, i, 0)),
            pl.BlockSpec((blk, d), lambda i: (i, 0)),
            pl.BlockSpec((d, d), lambda i: (0, 0)),
            pl.BlockSpec((blk, 1), lambda i: (i, 0)),
            pl.BlockSpec((1, 1, 2), lambda i: (i, 0, 0), memory_space=pltpu.SMEM),
            pl.BlockSpec((d, h2), lambda i: (0, 0)),
            pl.BlockSpec((1, h2), lambda i: (0, 0)),
            pl.BlockSpec((h2, nc), lambda i: (0, 0)),
            pl.BlockSpec((1, nc), lambda i: (0, 0)),
        ],
        out_specs=pl.BlockSpec((g, nc), lambda i: (0, 0)),
        out_shape=jax.ShapeDtypeStruct((g, nc), jnp.float32),
        scratch_shapes=[pltpu.VMEM((g, d), jnp.float32)],
    )(p.reshape(2, n2, d), r, Wr, batch_col, bounds,
      W4, b4.reshape(1, h2), W5, b5.reshape(1, nc))


def kernel(x, edge_index, batch, Wr1, br1, Wo1, Wr2, br2, Wo2, Wr3, br3, Wo3,
           W4, b4, W5, b5):
    n, d = x.shape
    src = edge_index[0].astype(jnp.int32)
    dst = edge_index[1].astype(jnp.int32)
    batch = batch.astype(jnp.int32)
    zeros = jnp.zeros((n, d), jnp.float32)
    blk = 1000
    bounds = jnp.stack([batch[::blk], batch[blk - 1::blk]], axis=1).reshape(-1, 1, 2)
    batch_col = batch.reshape(n, 1)

    r1 = _tc_root(x, Wo1, br1)
    p1 = _sc_agg(x, src, dst, zeros)
    h1, r2 = _tc_combine_root(p1, r1, Wr1, Wo2, br2, n)
    p2 = _sc_agg(h1, src, dst, zeros)
    h2, r3 = _tc_combine_root(p2, r2, Wr2, Wo3, br3, n)
    p3 = _sc_agg(h2, src, dst, zeros)
    return _tc_final(p3, r3, Wr3, batch_col, bounds, W4, b4, W5, b5, n)


# speedup vs baseline: 1.0963x; 1.0163x over previous
import functools

import jax
import jax.numpy as jnp
from jax import lax
from jax.experimental import pallas as pl
from jax.experimental.pallas import tpu as pltpu
from jax.experimental.pallas import tpu_sc as plsc

_N_GRAPHS = 64


def _sc_agg(h, src, dst, zeros):
    n, d = h.shape
    e = src.shape[0]
    nw = 32
    epw = e // nw
    chunk = 112
    n_main = epw // chunk
    etail = epw - n_main * chunk
    n_loop = (n_main - 5) // 3
    assert chunk % 16 == 0 and n_main == 3 * n_loop + 5 and etail % 8 == 0
    rpt = ((n // 16 + 7) // 8) * 8
    rtail = n - 15 * rpt

    mesh = plsc.VectorSubcoreMesh(core_axis_name="c", subcore_axis_name="s")

    vm = pltpu.VMEM
    scratch = []
    for _ in range(3):
        scratch += [vm((chunk,), jnp.int32),
                    vm((chunk,), jnp.int32),
                    vm((chunk,), jnp.int32),
                    vm((chunk, d), jnp.float32),
                    pltpu.SemaphoreType.DMA,
                    pltpu.SemaphoreType.DMA,
                    pltpu.SemaphoreType.DMA]
    scratch.append(vm((etail,), jnp.int32))
    scratch.append(pltpu.VMEM_SHARED((n, d), jnp.float32))

    @functools.partial(
        pl.kernel,
        out_type=jax.ShapeDtypeStruct((2 * n, d), jnp.float32),
        mesh=mesh,
        scratch_types=scratch,
    )
    def agg_kernel(h_hbm, src_hbm, dst_hbm, z_hbm, out_hbm, *rest):
        acc = rest[-1]
        didx_t = rest[-2]
        slots = [rest[7 * s:7 * s + 7] for s in range(3)]
        cid = lax.axis_index("c")
        sid = lax.axis_index("s")
        wid = sid * 2 + cid
        base = wid * epw

        def start_idx(s, ci):
            off = base + ci * chunk
            pltpu.async_copy(src_hbm.at[pl.ds(off, chunk)], slots[s][0], slots[s][4])
            pltpu.async_copy(dst_hbm.at[pl.ds(off, chunk)], slots[s][1], slots[s][4])

        def wait_idx(s):
            pltpu.make_async_copy(src_hbm.at[pl.ds(0, chunk)], slots[s][0],
                                  slots[s][4]).wait()
            pltpu.make_async_copy(dst_hbm.at[pl.ds(0, chunk)], slots[s][1],
                                  slots[s][4]).wait()

        def start_gather(s):
            pltpu.async_copy(h_hbm.at[slots[s][0]], slots[s][3], slots[s][5])

        def wait_gather(s):
            pltpu.make_async_copy(h_hbm.at[slots[s][0]], slots[s][3],
                                  slots[s][5]).wait()

        def start_scatter(s):
            pltpu.async_copy(slots[s][3], acc.at[slots[s][2]], slots[s][6],
                             add=True)

        def wait_scatter(s):
            pltpu.make_async_copy(slots[s][3], acc.at[slots[s][2]],
                                  slots[s][6]).wait()

        def step(c, s, wait_sc, wait_g_prev, prefetch, scatter_prev):
            prev = (s - 1) % 3
            wait_idx(s)
            if wait_sc:
                wait_scatter(s)
            for k in range(chunk // 16):
                slots[s][2][pl.ds(16 * k, 16)] = slots[s][1][pl.ds(16 * k, 16)]
            start_gather(s)
            if wait_g_prev:
                wait_gather(prev)
            if prefetch:
                start_idx((s + 2) % 3, c + 2)
            if scatter_prev:
                start_scatter(prev)

        @pl.when(sid < 15)
        def _():
            pltpu.sync_copy(z_hbm.at[pl.ds(sid * rpt, rpt)],
                            acc.at[pl.ds(sid * rpt, rpt)])

        @pl.when(sid == 15)
        def _():
            pltpu.sync_copy(z_hbm.at[pl.ds(15 * rpt, rtail)],
                            acc.at[pl.ds(15 * rpt, rtail)])

        plsc.subcore_barrier()

        start_idx(0, 0)
        start_idx(1, 1)
        step(0, 0, False, False, True, False)
        step(1, 1, False, True, True, True)
        step(2, 2, False, True, True, True)

        def body(i, carry):
            c0 = 3 * i
            step(c0 + 0, 0, True, True, True, True)
            step(c0 + 1, 1, True, True, True, True)
            step(c0 + 2, 2, True, True, True, True)
            return carry

        lax.fori_loop(1, n_loop + 1, body, 0)
        c0 = 3 * (n_loop + 1)
        step(c0 + 0, 0, True, True, False, True)
        step(c0 + 1, 1, True, True, False, True)
        wait_gather(1)
        start_scatter(1)
        toff = base + n_main * chunk
        wait_scatter(2)
        wait_scatter(0)
        pltpu.sync_copy(src_hbm.at[pl.ds(toff, etail)],
                        slots[0][0].at[pl.ds(0, etail)])
        pltpu.sync_copy(dst_hbm.at[pl.ds(toff, etail)], didx_t)
        pltpu.async_copy(h_hbm.at[slots[0][0].at[pl.ds(0, etail)]],
                         slots[0][3].at[pl.ds(0, etail), :], slots[0][5])
        pltpu.make_async_copy(h_hbm.at[slots[0][0].at[pl.ds(0, etail)]],
                              slots[0][3].at[pl.ds(0, etail), :],
                              slots[0][5]).wait()
        pltpu.sync_copy(slots[0][3].at[pl.ds(0, etail), :], acc.at[didx_t],
                        add=True)
        wait_scatter(1)

        plsc.subcore_barrier()

        @pl.when(sid < 15)
        def _():
            pltpu.sync_copy(acc.at[pl.ds(sid * rpt, rpt)],
                            out_hbm.at[pl.ds(cid * n + sid * rpt, rpt)])

        @pl.when(sid == 15)
        def _():
            pltpu.sync_copy(acc.at[pl.ds(15 * rpt, rtail)],
                            out_hbm.at[pl.ds(cid * n + 15 * rpt, rtail)])

    return agg_kernel(h, src, dst, zeros)


def _tc_root(h, Wo, br):
    n, d = h.shape
    blk = 1000
    nblk = n // blk

    def body(h_ref, wo_ref, br_ref, out_ref):
        out_ref[...] = jnp.dot(h_ref[...], wo_ref[...],
                               preferred_element_type=jnp.float32) + br_ref[...]

    return pl.pallas_call(
        body,
        grid=(nblk,),
        in_specs=[
            pl.BlockSpec((blk, d), lambda i: (i, 0)),
            pl.BlockSpec((d, d), lambda i: (0, 0)),
            pl.BlockSpec((1, d), lambda i: (0, 0)),
        ],
        out_specs=pl.BlockSpec((blk, d), lambda i: (i, 0)),
        out_shape=jax.ShapeDtypeStruct((n, d), jnp.float32),
    )(h, Wo, br.reshape(1, d))


def _tc_combine(p, r, Wr, n2):
    n, d = r.shape
    blk = 1000
    nblk = n // blk

    def body(p_ref, r_ref, wr_ref, out_ref):
        s = p_ref[0] + p_ref[1]
        acc = jnp.dot(s, wr_ref[...], preferred_element_type=jnp.float32)
        out_ref[...] = jnp.maximum(acc + r_ref[...], 0.0)

    return pl.pallas_call(
        body,
        grid=(nblk,),
        in_specs=[
            pl.BlockSpec((2, blk, d), lambda i: (0, i, 0)),
            pl.BlockSpec((blk, d), lambda i: (i, 0)),
            pl.BlockSpec((d, d), lambda i: (0, 0)),
        ],
        out_specs=pl.BlockSpec((blk, d), lambda i: (i, 0)),
        out_shape=jax.ShapeDtypeStruct((n, d), jnp.float32),
    )(p.reshape(2, n2, d), r, Wr)


def _tc_final(p, r, Wr, batch_col, bounds, W4, b4, W5, b5, n2):
    n, d = r.shape
    blk = 1000
    nblk = n // blk
    g = _N_GRAPHS
    h2 = W4.shape[1]
    nc = W5.shape[1]
    neg_inf = float("-inf")

    def body(p_ref, r_ref, wr_ref, bc_ref, bd_ref,
             w4_ref, b4_ref, w5_ref, b5_ref, out_ref, pooled):
        i = pl.program_id(0)
        acc = jnp.dot(p_ref[0] + p_ref[1], wr_ref[...],
                      preferred_element_type=jnp.float32)
        h3 = jnp.maximum(acc + r_ref[...], 0.0)

        @pl.when(i == 0)
        def _():
            pooled[...] = jnp.full((g, d), neg_inf, jnp.float32)

        g0 = bd_ref[0, 0, 0]
        g1 = bd_ref[0, 0, 1]

        def gbody(gi, carry):
            m = bc_ref[...] == gi
            cur = jnp.max(jnp.where(m, h3, neg_inf), axis=0, keepdims=True)
            pooled[pl.ds(gi, 1), :] = jnp.maximum(pooled[pl.ds(gi, 1), :], cur)
            return carry

        lax.fori_loop(g0, g1 + 1, gbody, 0)

        @pl.when(i == nblk - 1)
        def _():
            t = jnp.maximum(
                jnp.dot(pooled[...], w4_ref[...], preferred_element_type=jnp.float32)
                + b4_ref[...], 0.0)
            z = jnp.dot(t, w5_ref[...], preferred_element_type=jnp.float32) + b5_ref[...]
            mz = jnp.max(z, axis=-1, keepdims=True)
            ez = jnp.exp(z - mz)
            out_ref[...] = z - mz - jnp.log(jnp.sum(ez, axis=-1, keepdims=True))

    return pl.pallas_call(
        body,
        grid=(nblk,),
        in_specs=[
            pl.BlockSpec((2, blk, d), lambda i: (0, i, 0)),
            pl.BlockSpec((blk, d), lambda i: (i, 0)),
            pl.BlockSpec((d, d), lambda i: (0, 0)),
            pl.BlockSpec((blk, 1), lambda i: (i, 0)),
            pl.BlockSpec((1, 1, 2), lambda i: (i, 0, 0), memory_space=pltpu.SMEM),
            pl.BlockSpec((d, h2), lambda i: (0, 0)),
            pl.BlockSpec((1, h2), lambda i: (0, 0)),
            pl.BlockSpec((h2, nc), lambda i: (0, 0)),
            pl.BlockSpec((1, nc), lambda i: (0, 0)),
        ],
        out_specs=pl.BlockSpec((g, nc), lambda i: (0, 0)),
        out_shape=jax.ShapeDtypeStruct((g, nc), jnp.float32),
        scratch_shapes=[pltpu.VMEM((g, d), jnp.float32)],
    )(p.reshape(2, n2, d), r, Wr, batch_col, bounds,
      W4, b4.reshape(1, h2), W5, b5.reshape(1, nc))


def kernel(x, edge_index, batch, Wr1, br1, Wo1, Wr2, br2, Wo2, Wr3, br3, Wo3,
           W4, b4, W5, b5):
    n, d = x.shape
    src = edge_index[0].astype(jnp.int32)
    dst = edge_index[1].astype(jnp.int32)
    batch = batch.astype(jnp.int32)
    zeros = jnp.zeros((n, d), jnp.float32)
    blk = 1000
    bounds = jnp.stack([batch[::blk], batch[blk - 1::blk]], axis=1).reshape(-1, 1, 2)
    batch_col = batch.reshape(n, 1)

    r1 = _tc_root(x, Wo1, br1)
    p1 = _sc_agg(x, src, dst, zeros)
    h1 = _tc_combine(p1, r1, Wr1, n)
    r2 = _tc_root(h1, Wo2, br2)
    p2 = _sc_agg(h1, src, dst, zeros)
    h2 = _tc_combine(p2, r2, Wr2, n)
    r3 = _tc_root(h2, Wo3, br3)
    p3 = _sc_agg(h2, src, dst, zeros)
    return _tc_final(p3, r3, Wr3, batch_col, bounds, W4, b4, W5, b5, n)
